# Initial kernel scaffold; baseline (speedup 1.0000x reference)
import jax
import jax.numpy as jnp
from jax.experimental import pallas as pl


def kernel(edge_logits, edge_feats, node_feats, edge_index, W_e, b_e, w_ih, w_hh, b_ih, b_hh):
    raise NotImplementedError("write your pallas kernel here")


# speedup vs baseline: 13.4775x; 13.4775x over previous
import functools

import jax
import jax.numpy as jnp
from jax import lax
from jax.experimental import pallas as pl
from jax.experimental.pallas import tpu as pltpu
from jax.experimental.pallas import tpu_sc as plsc

N_NODES = 10000
N_EDGES = 320000
D_EDGE = 16
D_HID = 128
D_NODE = 128

NC = 2
NS = 16
NW = NC * NS
EPT = 10240
E_PAD = NW * EPT
CHUNK = 1024
NCHUNK = EPT // CHUNK
N_PAD = 10240
ROWS_PER_TILE = N_PAD // NS


def _sc_body(dst_hbm, log_hbm, feats_hbm, outT, outD, dst_v, ex_v, f_v, den_v, T_sh):
    c = lax.axis_index("c")
    s = lax.axis_index("s")
    wid = c * NS + s
    z16 = jnp.zeros((16,), jnp.float32)

    def zden(i, carry):
        den_v[pl.ds(i * 16, 16)] = z16
        return carry
    lax.fori_loop(0, N_PAD // 16, zden, 0)

    def zf(i, carry):
        f_v[i, :] = z16
        return carry
    lax.fori_loop(0, ROWS_PER_TILE, zf, 0)
    pltpu.sync_copy(f_v.at[pl.ds(0, ROWS_PER_TILE)],
                    T_sh.at[pl.ds(s * ROWS_PER_TILE, ROWS_PER_TILE)])
    plsc.subcore_barrier()

    pltpu.sync_copy(dst_hbm.at[wid], dst_v)
    pltpu.sync_copy(log_hbm.at[wid], ex_v)

    def exden(r, carry):
        for cc in range(8):
            off = r * 128 + cc * 16
            dv = dst_v[r, pl.ds(cc * 16, 16)]
            ev = jnp.exp(ex_v[pl.ds(off, 16)])
            ex_v[pl.ds(off, 16)] = ev
            plsc.addupdate_scatter(den_v, [dv], ev)
        return carry
    lax.fori_loop(0, EPT // 128, exden, 0)

    def chunk(k, carry):
        pltpu.sync_copy(feats_hbm.at[pl.ds(wid * EPT + k * CHUNK, CHUNK)], f_v)

        def scale(g, carry2):
            ex16 = ex_v[pl.ds(k * CHUNK + g * 16, 16)]
            for l in range(16):
                j = g * 16 + l
                f_v[j, :] = f_v[j, :] * jnp.full((16,), ex16[l], jnp.float32)
            return carry2
        lax.fori_loop(0, CHUNK // 16, scale, 0)

        for j2 in range(CHUNK // 128):
            pltpu.sync_copy(f_v.at[pl.ds(j2 * 128, 128)],
                            T_sh.at[dst_v.at[k * (CHUNK // 128) + j2]],
                            add=True)
        return carry
    lax.fori_loop(0, NCHUNK, chunk, 0)

    pltpu.sync_copy(den_v, outD.at[wid])
    plsc.subcore_barrier()
    pltpu.sync_copy(T_sh.at[pl.ds(s * ROWS_PER_TILE, ROWS_PER_TILE)],
                    outT.at[c].at[pl.ds(s * ROWS_PER_TILE, ROWS_PER_TILE)])


_sc_agg = functools.partial(
    pl.kernel,
    out_type=[
        jax.ShapeDtypeStruct((NC, N_PAD, D_EDGE), jnp.float32),
        jax.ShapeDtypeStruct((NW, N_PAD), jnp.float32),
    ],
    mesh=plsc.VectorSubcoreMesh(core_axis_name="c", subcore_axis_name="s"),
    compiler_params=pltpu.CompilerParams(needs_layout_passes=False,
                                         use_tc_tiling_on_sc=False),
    scratch_types=[
        pltpu.VMEM((EPT // 128, 128), jnp.int32),
        pltpu.VMEM((EPT,), jnp.float32),
        pltpu.VMEM((CHUNK, D_EDGE), jnp.float32),
        pltpu.VMEM((N_PAD,), jnp.float32),
        pltpu.VMEM_SHARED((N_PAD, D_EDGE), jnp.float32),
    ],
)(_sc_body)


BLK = 1024


def _tc_body(T_ref, d_ref, nf_ref, wet_ref, be_ref, wiht_ref, whht_ref,
             bih_ref, bhh_ref, o_ref):
    T = T_ref[0] + T_ref[1]
    ones = jnp.ones((NW, 1), jnp.float32)
    den = lax.dot_general(d_ref[...], ones, (((0,), (0,)), ((), ())),
                          preferred_element_type=jnp.float32)
    has = den > 0.0
    dsafe = jnp.where(has, den, 1.0)
    S = T / dsafe
    cpre = jnp.dot(S, wet_ref[...], preferred_element_type=jnp.float32)
    cpre = cpre + jnp.where(has, 1.0, 0.0) * be_ref[...]
    ctx = jnp.where(cpre > 0.0, cpre, jnp.exp(jnp.minimum(cpre, 0.0)) - 1.0)
    gi = jnp.dot(ctx, wiht_ref[...], preferred_element_type=jnp.float32) + bih_ref[...]
    nf = nf_ref[...]
    gh = jnp.dot(nf, whht_ref[...], preferred_element_type=jnp.float32) + bhh_ref[...]
    r = jax.nn.sigmoid(gi[:, 0:D_NODE] + gh[:, 0:D_NODE])
    zg = jax.nn.sigmoid(gi[:, D_NODE:2 * D_NODE] + gh[:, D_NODE:2 * D_NODE])
    n = jnp.tanh(gi[:, 2 * D_NODE:] + r * gh[:, 2 * D_NODE:])
    h = (1.0 - zg) * n + zg * nf
    o_ref[...] = jnp.maximum(h, 0.0)


_tc_gru = pl.pallas_call(
    _tc_body,
    out_shape=jax.ShapeDtypeStruct((N_PAD, D_NODE), jnp.float32),
    grid=(N_PAD // BLK,),
    in_specs=[
        pl.BlockSpec((NC, BLK, D_EDGE), lambda i: (0, i, 0)),
        pl.BlockSpec((NW, BLK), lambda i: (0, i)),
        pl.BlockSpec((BLK, D_NODE), lambda i: (i, 0)),
        pl.BlockSpec((D_EDGE, D_HID), lambda i: (0, 0)),
        pl.BlockSpec((1, D_HID), lambda i: (0, 0)),
        pl.BlockSpec((D_HID, 3 * D_NODE), lambda i: (0, 0)),
        pl.BlockSpec((D_NODE, 3 * D_NODE), lambda i: (0, 0)),
        pl.BlockSpec((1, 3 * D_NODE), lambda i: (0, 0)),
        pl.BlockSpec((1, 3 * D_NODE), lambda i: (0, 0)),
    ],
    out_specs=pl.BlockSpec((BLK, D_NODE), lambda i: (i, 0)),
)


def kernel(edge_logits, edge_feats, node_feats, edge_index, W_e, b_e,
           w_ih, w_hh, b_ih, b_hh):
    dst = edge_index[1]
    pad = E_PAD - N_EDGES
    dst_p = jnp.concatenate([dst, jnp.zeros((pad,), jnp.int32)])
    log_p = jnp.concatenate([edge_logits[:, 0],
                             jnp.full((pad,), -1e30, jnp.float32)])
    feats_p = jnp.concatenate([edge_feats,
                               jnp.zeros((pad, D_EDGE), jnp.float32)])
    T, D = _sc_agg(dst_p.reshape(NW, EPT // 128, 128),
                   log_p.reshape(NW, EPT),
                   feats_p)
    nf_p = jnp.concatenate([node_feats,
                            jnp.zeros((N_PAD - N_NODES, D_NODE), jnp.float32)])
    h = _tc_gru(T, D, nf_p, W_e.T, b_e.reshape(1, -1),
                w_ih.T, w_hh.T, b_ih.reshape(1, -1), b_hh.reshape(1, -1))
    return h[:N_NODES]


# speedup vs baseline: 19.9592x; 1.4809x over previous
import functools

import jax
import jax.numpy as jnp
from jax import lax
from jax.experimental import pallas as pl
from jax.experimental.pallas import tpu as pltpu
from jax.experimental.pallas import tpu_sc as plsc

N_NODES = 10000
N_EDGES = 320000
D_EDGE = 16
D_HID = 128
D_NODE = 128

NC = 2
NS = 16
NW = NC * NS
EPT = 10240
E_PAD = NW * EPT
CHUNK = 2560
NCHUNK = EPT // CHUNK
ROWS_PER_TILE = N_NODES // NS
DBLK = 1000


def _sc_body(dst_hbm, log_hbm, feats_hbm, outT, outD, dst_v, ex_v, f_v, den_v, T_sh):
    c = lax.axis_index("c")
    s = lax.axis_index("s")
    wid = c * NS + s
    z16 = jnp.zeros((16,), jnp.float32)

    def zden(i, carry):
        den_v[pl.ds(i * 16, 16)] = z16
        return carry
    lax.fori_loop(0, N_NODES // 16, zden, 0)

    def zf(i, carry):
        f_v[i, :] = z16
        return carry
    lax.fori_loop(0, ROWS_PER_TILE, zf, 0)
    pltpu.sync_copy(f_v.at[pl.ds(0, ROWS_PER_TILE)],
                    T_sh.at[pl.ds(s * ROWS_PER_TILE, ROWS_PER_TILE)])
    plsc.subcore_barrier()

    pltpu.sync_copy(dst_hbm.at[wid], dst_v)
    pltpu.sync_copy(log_hbm.at[wid], ex_v)

    def exden(r, carry):
        for cc in range(8):
            off = r * 128 + cc * 16
            dv = dst_v[r, pl.ds(cc * 16, 16)]
            ev = jnp.exp(ex_v[pl.ds(off, 16)])
            ex_v[pl.ds(off, 16)] = ev
            plsc.addupdate_scatter(den_v, [dv], ev)
        return carry
    lax.fori_loop(0, EPT // 128, exden, 0)

    def chunk(k, carry):
        off = wid * EPT + k * CHUNK
        offc = jnp.minimum(off, N_EDGES - CHUNK)
        pltpu.sync_copy(feats_hbm.at[pl.ds(offc, CHUNK)], f_v)

        def scale(g, carry2):
            ex16 = ex_v[pl.ds(k * CHUNK + g * 16, 16)]
            for l in range(16):
                j = g * 16 + l
                f_v[j, :] = f_v[j, :] * jnp.full((16,), ex16[l], jnp.float32)
            return carry2
        lax.fori_loop(0, CHUNK // 16, scale, 0)

        for j2 in range(CHUNK // 128):
            pltpu.sync_copy(f_v.at[pl.ds(j2 * 128, 128)],
                            T_sh.at[dst_v.at[k * (CHUNK // 128) + j2]],
                            add=True)
        return carry
    lax.fori_loop(0, NCHUNK, chunk, 0)

    for j in range(N_NODES // DBLK):
        pltpu.sync_copy(den_v.at[pl.ds(j * DBLK, DBLK)], outD.at[j].at[wid])
    plsc.subcore_barrier()
    pltpu.sync_copy(T_sh.at[pl.ds(s * ROWS_PER_TILE, ROWS_PER_TILE)],
                    outT.at[c].at[pl.ds(s * ROWS_PER_TILE, ROWS_PER_TILE)])


_sc_agg = functools.partial(
    pl.kernel,
    out_type=[
        jax.ShapeDtypeStruct((NC, N_NODES, D_EDGE), jnp.float32),
        jax.ShapeDtypeStruct((N_NODES // DBLK, NW, DBLK), jnp.float32),
    ],
    mesh=plsc.VectorSubcoreMesh(core_axis_name="c", subcore_axis_name="s"),
    compiler_params=pltpu.CompilerParams(needs_layout_passes=False,
                                         use_tc_tiling_on_sc=False),
    scratch_types=[
        pltpu.VMEM((EPT // 128, 128), jnp.int32),
        pltpu.VMEM((EPT,), jnp.float32),
        pltpu.VMEM((CHUNK, D_EDGE), jnp.float32),
        pltpu.VMEM((N_NODES,), jnp.float32),
        pltpu.VMEM_SHARED((N_NODES, D_EDGE), jnp.float32),
    ],
)(_sc_body)


BLK = 1000


def _tc_body(T_ref, d_ref, nf_ref, wet_ref, be_ref, wiht_ref, whht_ref,
             bih_ref, bhh_ref, o_ref):
    T = T_ref[0] + T_ref[1]
    ones = jnp.ones((NW, 1), jnp.float32)
    den = lax.dot_general(d_ref[0], ones, (((0,), (0,)), ((), ())),
                          preferred_element_type=jnp.float32)
    has = den > 0.0
    dsafe = jnp.where(has, den, 1.0)
    S = T / dsafe
    cpre = jnp.dot(S, wet_ref[...], preferred_element_type=jnp.float32)
    cpre = cpre + jnp.where(has, 1.0, 0.0) * be_ref[...]
    ctx = jnp.where(cpre > 0.0, cpre, jnp.exp(jnp.minimum(cpre, 0.0)) - 1.0)
    gi = jnp.dot(ctx, wiht_ref[...], preferred_element_type=jnp.float32) + bih_ref[...]
    nf = nf_ref[...]
    gh = jnp.dot(nf, whht_ref[...], preferred_element_type=jnp.float32) + bhh_ref[...]
    r = jax.nn.sigmoid(gi[:, 0:D_NODE] + gh[:, 0:D_NODE])
    zg = jax.nn.sigmoid(gi[:, D_NODE:2 * D_NODE] + gh[:, D_NODE:2 * D_NODE])
    n = jnp.tanh(gi[:, 2 * D_NODE:] + r * gh[:, 2 * D_NODE:])
    h = (1.0 - zg) * n + zg * nf
    o_ref[...] = jnp.maximum(h, 0.0)


_tc_gru = pl.pallas_call(
    _tc_body,
    out_shape=jax.ShapeDtypeStruct((N_NODES, D_NODE), jnp.float32),
    grid=(N_NODES // BLK,),
    in_specs=[
        pl.BlockSpec((NC, BLK, D_EDGE), lambda i: (0, i, 0)),
        pl.BlockSpec((1, NW, DBLK), lambda i: (i, 0, 0)),
        pl.BlockSpec((BLK, D_NODE), lambda i: (i, 0)),
        pl.BlockSpec((D_EDGE, D_HID), lambda i: (0, 0)),
        pl.BlockSpec((1, D_HID), lambda i: (0, 0)),
        pl.BlockSpec((D_HID, 3 * D_NODE), lambda i: (0, 0)),
        pl.BlockSpec((D_NODE, 3 * D_NODE), lambda i: (0, 0)),
        pl.BlockSpec((1, 3 * D_NODE), lambda i: (0, 0)),
        pl.BlockSpec((1, 3 * D_NODE), lambda i: (0, 0)),
    ],
    out_specs=pl.BlockSpec((BLK, D_NODE), lambda i: (i, 0)),
)


def kernel(edge_logits, edge_feats, node_feats, edge_index, W_e, b_e,
           w_ih, w_hh, b_ih, b_hh):
    dst = edge_index[1]
    pad = E_PAD - N_EDGES
    dst_p = jnp.concatenate([dst, jnp.zeros((pad,), jnp.int32)])
    log_p = jnp.concatenate([edge_logits[:, 0],
                             jnp.full((pad,), -1e30, jnp.float32)])
    T, D = _sc_agg(dst_p.reshape(NW, EPT // 128, 128),
                   log_p.reshape(NW, EPT),
                   edge_feats)
    return _tc_gru(T, D, node_feats, W_e.T, b_e.reshape(1, -1),
                   w_ih.T, w_hh.T, b_ih.reshape(1, -1), b_hh.reshape(1, -1))


# speedup vs baseline: 20.9453x; 1.0494x over previous
import functools

import jax
import jax.numpy as jnp
from jax import lax
from jax.experimental import pallas as pl
from jax.experimental.pallas import tpu as pltpu
from jax.experimental.pallas import tpu_sc as plsc

N_NODES = 10000
N_EDGES = 320000
D_EDGE = 16
D_HID = 128
D_NODE = 128

NC = 2
NS = 16
NW = NC * NS
ROWS_T = N_EDGES // 128
RPW = 80
CROWS = 20
CHUNK = CROWS * 128
ROWS_PER_TILE = N_NODES // NS
DBLK = 1000


def _sc_body(ei_hbm, lg_hbm, feats_hbm, outT, outD, dst_v, ex_v, f_v, den_v, T_sh):
    c = lax.axis_index("c")
    s = lax.axis_index("s")
    wid = c * NS + s
    z16 = jnp.zeros((16,), jnp.float32)

    row0 = wid * RPW
    row0c = jnp.minimum(row0, ROWS_T - RPW)
    loc = row0 - row0c
    nr = jnp.minimum(RPW, ROWS_T - row0)

    def zden(i, carry):
        den_v[pl.ds(i * 16, 16)] = z16
        return carry
    lax.fori_loop(0, N_NODES // 16, zden, 0)

    def zf(i, carry):
        f_v[i, :] = z16
        return carry
    lax.fori_loop(0, ROWS_PER_TILE, zf, 0)
    pltpu.sync_copy(f_v.at[pl.ds(0, ROWS_PER_TILE)],
                    T_sh.at[pl.ds(s * ROWS_PER_TILE, ROWS_PER_TILE)])
    plsc.subcore_barrier()

    pltpu.sync_copy(ei_hbm.at[1].at[pl.ds(row0c, RPW)], dst_v)
    pltpu.sync_copy(lg_hbm.at[pl.ds(row0c, RPW)], ex_v)

    def exden(r, carry):
        for cc in range(8):
            dv = dst_v[r, pl.ds(cc * 16, 16)]
            ev = jnp.exp(ex_v[r, pl.ds(cc * 16, 16)])
            ex_v[r, pl.ds(cc * 16, 16)] = ev
            plsc.addupdate_scatter(den_v, [dv], ev)
        return carry
    lax.fori_loop(loc, loc + nr, exden, 0)

    def chunk(k, carry):
        pltpu.sync_copy(feats_hbm.at[pl.ds((row0 + k * CROWS) * 128, CHUNK)], f_v)

        def scale(rl, carry2):
            r = loc + k * CROWS + rl
            for cc in range(8):
                ex16 = ex_v[r, pl.ds(cc * 16, 16)]
                for l in range(16):
                    j = rl * 128 + cc * 16 + l
                    f_v[j, :] = f_v[j, :] * jnp.full((16,), ex16[l], jnp.float32)
            return carry2
        lax.fori_loop(0, CROWS, scale, 0)

        for j2 in range(CROWS):
            pltpu.sync_copy(f_v.at[pl.ds(j2 * 128, 128)],
                            T_sh.at[dst_v.at[loc + k * CROWS + j2]],
                            add=True)
        return carry
    lax.fori_loop(0, nr // CROWS, chunk, 0)

    for j in range(N_NODES // DBLK):
        pltpu.sync_copy(den_v.at[pl.ds(j * DBLK, DBLK)], outD.at[j].at[wid])
    plsc.subcore_barrier()
    pltpu.sync_copy(T_sh.at[pl.ds(s * ROWS_PER_TILE, ROWS_PER_TILE)],
                    outT.at[c].at[pl.ds(s * ROWS_PER_TILE, ROWS_PER_TILE)])


_sc_agg = functools.partial(
    pl.kernel,
    out_type=[
        jax.ShapeDtypeStruct((NC, N_NODES, D_EDGE), jnp.float32),
        jax.ShapeDtypeStruct((N_NODES // DBLK, NW, DBLK), jnp.float32),
    ],
    mesh=plsc.VectorSubcoreMesh(core_axis_name="c", subcore_axis_name="s"),
    compiler_params=pltpu.CompilerParams(needs_layout_passes=False,
                                         use_tc_tiling_on_sc=False),
    scratch_types=[
        pltpu.VMEM((RPW, 128), jnp.int32),
        pltpu.VMEM((RPW, 128), jnp.float32),
        pltpu.VMEM((CHUNK, D_EDGE), jnp.float32),
        pltpu.VMEM((N_NODES,), jnp.float32),
        pltpu.VMEM_SHARED((N_NODES, D_EDGE), jnp.float32),
    ],
)(_sc_body)


BLK = 1000


def _tc_body(T_ref, d_ref, nf_ref, wet_ref, be_ref, wiht_ref, whht_ref,
             bih_ref, bhh_ref, o_ref):
    T = T_ref[0] + T_ref[1]
    ones = jnp.ones((NW, 1), jnp.float32)
    den = lax.dot_general(d_ref[0], ones, (((0,), (0,)), ((), ())),
                          preferred_element_type=jnp.float32)
    has = den > 0.0
    dsafe = jnp.where(has, den, 1.0)
    S = T / dsafe
    cpre = jnp.dot(S, wet_ref[...], preferred_element_type=jnp.float32)
    cpre = cpre + jnp.where(has, 1.0, 0.0) * be_ref[...]
    ctx = jnp.where(cpre > 0.0, cpre, jnp.exp(jnp.minimum(cpre, 0.0)) - 1.0)
    gi = jnp.dot(ctx, wiht_ref[...], preferred_element_type=jnp.float32) + bih_ref[...]
    nf = nf_ref[...]
    gh = jnp.dot(nf, whht_ref[...], preferred_element_type=jnp.float32) + bhh_ref[...]
    r = jax.nn.sigmoid(gi[:, 0:D_NODE] + gh[:, 0:D_NODE])
    zg = jax.nn.sigmoid(gi[:, D_NODE:2 * D_NODE] + gh[:, D_NODE:2 * D_NODE])
    n = jnp.tanh(gi[:, 2 * D_NODE:] + r * gh[:, 2 * D_NODE:])
    h = (1.0 - zg) * n + zg * nf
    o_ref[...] = jnp.maximum(h, 0.0)


_tc_gru = pl.pallas_call(
    _tc_body,
    out_shape=jax.ShapeDtypeStruct((N_NODES, D_NODE), jnp.float32),
    grid=(N_NODES // BLK,),
    in_specs=[
        pl.BlockSpec((NC, BLK, D_EDGE), lambda i: (0, i, 0)),
        pl.BlockSpec((1, NW, DBLK), lambda i: (i, 0, 0)),
        pl.BlockSpec((BLK, D_NODE), lambda i: (i, 0)),
        pl.BlockSpec((D_EDGE, D_HID), lambda i: (0, 0)),
        pl.BlockSpec((1, D_HID), lambda i: (0, 0)),
        pl.BlockSpec((D_HID, 3 * D_NODE), lambda i: (0, 0)),
        pl.BlockSpec((D_NODE, 3 * D_NODE), lambda i: (0, 0)),
        pl.BlockSpec((1, 3 * D_NODE), lambda i: (0, 0)),
        pl.BlockSpec((1, 3 * D_NODE), lambda i: (0, 0)),
    ],
    out_specs=pl.BlockSpec((BLK, D_NODE), lambda i: (i, 0)),
)


def kernel(edge_logits, edge_feats, node_feats, edge_index, W_e, b_e,
           w_ih, w_hh, b_ih, b_hh):
    T, D = _sc_agg(edge_index.reshape(2, ROWS_T, 128),
                   edge_logits.reshape(ROWS_T, 128),
                   edge_feats)
    return _tc_gru(T, D, node_feats, W_e.T, b_e.reshape(1, -1),
                   w_ih.T, w_hh.T, b_ih.reshape(1, -1), b_hh.reshape(1, -1))
